# baseline (speedup 1.0000x reference)
import jax
import jax.numpy as jnp
from jax.experimental import pallas as pl
from jax.experimental.pallas import tpu as pltpu

_CH = 16
_EMB = 128
_HB = 192
_W = 384


def _fused_kernel(xt, xc, xb, fm, mle, mre, mleb, mreb,
                  w1, b1, w2, b2, w3, b3, out):
    w = _W
    n = _HB * w
    nh = pl.num_programs(1)
    i = pl.program_id(1)
    tfac = jnp.where(i > 0, 1.0, 0.0)
    bfac = jnp.where(i < nh - 1, 1.0, 0.0)

    topb = (xt[0][:, 6:8, :].astype(jnp.bfloat16).reshape(_CH, 2 * w)
            * tfac.astype(jnp.bfloat16))
    botb = (xb[0][:, 0:2, :].astype(jnp.bfloat16).reshape(_CH, 2 * w)
            * bfac.astype(jnp.bfloat16))
    xeb = jnp.concatenate(
        [topb, xc[0].astype(jnp.bfloat16).reshape(_CH, n), botb], axis=1)

    mlb = mleb[...]
    mrb = mreb[...]

    xu = xeb[:, :n + 2 * w]
    xm = xeb[:, w:n + 3 * w]
    xd = xeb[:, 2 * w:]
    s = xu + 2.0 * xm + xd
    t = xd - xu

    ta = xt[0][3:4, 6:8, :].reshape(1, 2 * w) * tfac
    ba = xb[0][3:4, 0:2, :].reshape(1, 2 * w) * bfac
    ae = jnp.concatenate([ta, xc[0][3:4].reshape(1, n), ba], axis=1)
    pmf = jnp.maximum(jnp.maximum(ae[:, :n + 2 * w], ae[:, w:n + 3 * w]),
                      ae[:, 2 * w:])
    ml = mle[...]
    mr = mre[...]
    pooled = jnp.maximum(
        jnp.maximum(pmf[:, w + 1:w + 1 + n] * ml, pmf[:, w - 1:w - 1 + n] * mr),
        pmf[:, w:w + n])
    act = jnp.where((pooled > 0.1) & (fm[0].reshape(1, n) != 0), 1.0, 0.0)

    us = []
    n2 = n // 24
    for half in range(24):
        o = half * n2
        gx = (s[:, w + 1 + o:w + 1 + o + n2] * mlb[:, o:o + n2]
              - s[:, w - 1 + o:w - 1 + o + n2] * mrb[:, o:o + n2])
        gy = (t[:, w + 1 + o:w + 1 + o + n2] * mlb[:, o:o + n2]
              + t[:, w - 1 + o:w - 1 + o + n2] * mrb[:, o:o + n2]
              + 2.0 * t[:, w + o:w + o + n2])
        mid = xeb[:, 2 * w + o:2 * w + o + n2]
        f = jnp.concatenate([mid, gx, gy], axis=0)
        h1 = jnp.dot(w1[...], f, preferred_element_type=jnp.float32)
        h1 = jnp.maximum(h1.astype(jnp.bfloat16) + b1[...], 0)
        h2 = jnp.dot(w2[...], h1, preferred_element_type=jnp.float32)
        h2 = jnp.maximum(h2.astype(jnp.bfloat16) + b2[...], 0)
        us.append(jnp.dot(w3[...], h2, preferred_element_type=jnp.float32))
    u = jnp.concatenate(us, axis=1) + b3[...]
    out[0] = (u * act).reshape(_CH, _HB, w)


def kernel(x, fire_mask, W1, b1, W2, b2, W3, b3):
    B, C, H, W = x.shape
    nh = H // _HB
    n = _HB * W
    nhb = H // 8
    wcol = jnp.arange(n, dtype=jnp.int32) % W
    mle = (wcol != W - 1).astype(jnp.float32).reshape(1, n)
    mre = (wcol != 0).astype(jnp.float32).reshape(1, n)
    w1b = W1.astype(jnp.bfloat16)
    w2b = W2.astype(jnp.bfloat16)
    w3b = W3.astype(jnp.bfloat16)
    b1c = b1.astype(jnp.bfloat16).reshape(_EMB, 1)
    b2c = b2.astype(jnp.bfloat16).reshape(_EMB, 1)
    b3c = b3.reshape(_CH, 1)
    k = _HB // 8

    def const_spec(shape):
        return pl.BlockSpec(shape, lambda b, h: tuple(0 for _ in shape))

    return pl.pallas_call(
        _fused_kernel,
        grid=(B, nh),
        in_specs=[
            pl.BlockSpec((1, C, 8, W),
                         lambda b, h: (b, 0, jnp.maximum(k * h - 1, 0), 0)),
            pl.BlockSpec((1, C, _HB, W), lambda b, h: (b, 0, h, 0)),
            pl.BlockSpec((1, C, 8, W),
                         lambda b, h: (b, 0, jnp.minimum(k * (h + 1), nhb - 1), 0)),
            pl.BlockSpec((1, 1, _HB, W), lambda b, h: (b, 0, h, 0)),
            const_spec((1, n)),
            const_spec((1, n)),
            const_spec((1, n)),
            const_spec((1, n)),
            const_spec((_EMB, 3 * _CH)),
            const_spec((_EMB, 1)),
            const_spec((_EMB, _EMB)),
            const_spec((_EMB, 1)),
            const_spec((_CH, _EMB)),
            const_spec((_CH, 1)),
        ],
        out_specs=pl.BlockSpec((1, C, _HB, W), lambda b, h: (b, 0, h, 0)),
        out_shape=jax.ShapeDtypeStruct((B, C, H, W), jnp.float32),
        compiler_params=pltpu.CompilerParams(
            dimension_semantics=("parallel", "parallel")),
    )(x, x, x, fire_mask, mle, mre, mle.astype(jnp.bfloat16),
      mre.astype(jnp.bfloat16), w1b, b1c, w2b, b2c, w3b, b3c)


# speedup vs baseline: 1.0167x; 1.0167x over previous
import jax
import jax.numpy as jnp
from jax.experimental import pallas as pl
from jax.experimental.pallas import tpu as pltpu

_CH = 16
_EMB = 128
_HB = 192
_W = 384


def _fused_kernel(xt, xc, xb, fm, mle, mre, mleb, mreb,
                  w1, b1, w2, b2, w3, b3, out):
    w = _W
    n = _HB * w
    nh = pl.num_programs(1)
    i = pl.program_id(1)
    tfac = jnp.where(i > 0, 1.0, 0.0)
    bfac = jnp.where(i < nh - 1, 1.0, 0.0)

    topb = (xt[0][:, 6:8, :].astype(jnp.bfloat16).reshape(_CH, 2 * w)
            * tfac.astype(jnp.bfloat16))
    botb = (xb[0][:, 0:2, :].astype(jnp.bfloat16).reshape(_CH, 2 * w)
            * bfac.astype(jnp.bfloat16))
    xeb = jnp.concatenate(
        [topb, xc[0].astype(jnp.bfloat16).reshape(_CH, n), botb], axis=1)

    mlb = mleb[...]
    mrb = mreb[...]

    xu = xeb[:, :n + 2 * w]
    xm = xeb[:, w:n + 3 * w]
    xd = xeb[:, 2 * w:]
    s = xu + 2.0 * xm + xd
    t = xd - xu

    ta = xt[0][3:4, 6:8, :].reshape(1, 2 * w) * tfac
    ba = xb[0][3:4, 0:2, :].reshape(1, 2 * w) * bfac
    ae = jnp.concatenate([ta, xc[0][3:4].reshape(1, n), ba], axis=1)
    pmf = jnp.maximum(jnp.maximum(ae[:, :n + 2 * w], ae[:, w:n + 3 * w]),
                      ae[:, 2 * w:])
    ml = mle[...]
    mr = mre[...]
    pooled = jnp.maximum(
        jnp.maximum(pmf[:, w + 1:w + 1 + n] * ml, pmf[:, w - 1:w - 1 + n] * mr),
        pmf[:, w:w + n])
    act = jnp.where((pooled > 0.1) & (fm[0].reshape(1, n) != 0), 1.0, 0.0)

    us = []
    n2 = n // 12
    for half in range(12):
        o = half * n2
        gx = (s[:, w + 1 + o:w + 1 + o + n2] * mlb[:, o:o + n2]
              - s[:, w - 1 + o:w - 1 + o + n2] * mrb[:, o:o + n2])
        gy = (t[:, w + 1 + o:w + 1 + o + n2] * mlb[:, o:o + n2]
              + t[:, w - 1 + o:w - 1 + o + n2] * mrb[:, o:o + n2]
              + 2.0 * t[:, w + o:w + o + n2])
        mid = xeb[:, 2 * w + o:2 * w + o + n2]
        f = jnp.concatenate([mid, gx, gy], axis=0)
        h1 = jnp.dot(w1[...], f, preferred_element_type=jnp.float32)
        h1 = jnp.maximum(h1.astype(jnp.bfloat16) + b1[...], 0)
        h2 = jnp.dot(w2[...], h1, preferred_element_type=jnp.float32)
        h2 = jnp.maximum(h2.astype(jnp.bfloat16) + b2[...], 0)
        us.append(jnp.dot(w3[...], h2, preferred_element_type=jnp.float32))
    u = jnp.concatenate(us, axis=1) + b3[...]
    out[0] = (u * act).reshape(_CH, _HB, w)


def kernel(x, fire_mask, W1, b1, W2, b2, W3, b3):
    B, C, H, W = x.shape
    nh = H // _HB
    n = _HB * W
    nhb = H // 8
    wcol = jnp.arange(n, dtype=jnp.int32) % W
    mle = (wcol != W - 1).astype(jnp.float32).reshape(1, n)
    mre = (wcol != 0).astype(jnp.float32).reshape(1, n)
    w1b = W1.astype(jnp.bfloat16)
    w2b = W2.astype(jnp.bfloat16)
    w3b = W3.astype(jnp.bfloat16)
    b1c = b1.astype(jnp.bfloat16).reshape(_EMB, 1)
    b2c = b2.astype(jnp.bfloat16).reshape(_EMB, 1)
    b3c = b3.reshape(_CH, 1)
    k = _HB // 8

    def const_spec(shape):
        return pl.BlockSpec(shape, lambda b, h: tuple(0 for _ in shape))

    return pl.pallas_call(
        _fused_kernel,
        grid=(B, nh),
        in_specs=[
            pl.BlockSpec((1, C, 8, W),
                         lambda b, h: (b, 0, jnp.maximum(k * h - 1, 0), 0)),
            pl.BlockSpec((1, C, _HB, W), lambda b, h: (b, 0, h, 0)),
            pl.BlockSpec((1, C, 8, W),
                         lambda b, h: (b, 0, jnp.minimum(k * (h + 1), nhb - 1), 0)),
            pl.BlockSpec((1, 1, _HB, W), lambda b, h: (b, 0, h, 0)),
            const_spec((1, n)),
            const_spec((1, n)),
            const_spec((1, n)),
            const_spec((1, n)),
            const_spec((_EMB, 3 * _CH)),
            const_spec((_EMB, 1)),
            const_spec((_EMB, _EMB)),
            const_spec((_EMB, 1)),
            const_spec((_CH, _EMB)),
            const_spec((_CH, 1)),
        ],
        out_specs=pl.BlockSpec((1, C, _HB, W), lambda b, h: (b, 0, h, 0)),
        out_shape=jax.ShapeDtypeStruct((B, C, H, W), jnp.float32),
        compiler_params=pltpu.CompilerParams(
            dimension_semantics=("parallel", "parallel")),
    )(x, x, x, fire_mask, mle, mre, mle.astype(jnp.bfloat16),
      mre.astype(jnp.bfloat16), w1b, b1c, w2b, b2c, w3b, b3c)
